# Initial kernel scaffold; baseline (speedup 1.0000x reference)
import jax
import jax.numpy as jnp
from jax.experimental import pallas as pl


def kernel(x, atom_emb):
    raise NotImplementedError("write your pallas kernel here")


# speedup vs baseline: 3.3307x; 3.3307x over previous
import functools

import jax
import jax.numpy as jnp
from jax import lax
from jax.experimental import pallas as pl
from jax.experimental.pallas import tpu as pltpu
from jax.experimental.pallas import tpu_sc as plsc

EMB_D = 128
_info = plsc.get_sparse_core_info()
_NC, _NS = _info.num_cores, _info.num_subcores
_NW = _NC * _NS

_CHUNK = 512
_SUB = 128


def _make_gather(n_total: int):
    assert n_total % (_NW * _CHUNK) == 0
    bpw = n_total // _NW
    n_chunks = bpw // _CHUNK
    mesh = plsc.VectorSubcoreMesh(core_axis_name="c", subcore_axis_name="s")

    @functools.partial(
        pl.kernel,
        out_type=jax.ShapeDtypeStruct((n_total, EMB_D), jnp.float32),
        mesh=mesh,
        scratch_types=[
            pltpu.VMEM((_CHUNK,), jnp.int32),
            pltpu.VMEM((_CHUNK, EMB_D), jnp.float32),
            pltpu.SemaphoreType.DMA,
        ],
    )
    def gather_kernel(idx_hbm, table_hbm, out_hbm, idx_v, rows_v, sem):
        wid = lax.axis_index("s") * _NC + lax.axis_index("c")
        base = wid * bpw

        def chunk_body(i, carry):
            off = base + i * _CHUNK
            pltpu.sync_copy(idx_hbm.at[pl.ds(off, _CHUNK)], idx_v)
            copies = [
                pltpu.async_copy(
                    table_hbm.at[idx_v.at[pl.ds(j * _SUB, _SUB)]],
                    rows_v.at[pl.ds(j * _SUB, _SUB)],
                    sem,
                )
                for j in range(_CHUNK // _SUB)
            ]
            for c in copies:
                c.wait()
            pltpu.sync_copy(rows_v, out_hbm.at[pl.ds(off, _CHUNK)])
            return carry

        lax.fori_loop(0, n_chunks, chunk_body, 0)

    return gather_kernel


def kernel(x, atom_emb):
    b, s = x.shape
    flat_idx = x.reshape(-1).astype(jnp.int32)
    out = _make_gather(b * s)(flat_idx, atom_emb)
    return out.reshape(b, s, EMB_D)


# speedup vs baseline: 3.4496x; 1.0357x over previous
import functools

import jax
import jax.numpy as jnp
from jax import lax
from jax.experimental import pallas as pl
from jax.experimental.pallas import tpu as pltpu
from jax.experimental.pallas import tpu_sc as plsc

EMB_D = 128
_info = plsc.get_sparse_core_info()
_NC, _NS = _info.num_cores, _info.num_subcores
_NW = _NC * _NS

_CHUNK = 256
_SUB = 128


def _make_gather(n_total: int):
    assert n_total % (_NW * _CHUNK) == 0
    bpw = n_total // _NW
    n_chunks = bpw // _CHUNK
    assert n_chunks >= 4 and n_chunks % 2 == 0
    mesh = plsc.VectorSubcoreMesh(core_axis_name="c", subcore_axis_name="s")

    @functools.partial(
        pl.kernel,
        out_type=jax.ShapeDtypeStruct((n_total, EMB_D), jnp.float32),
        mesh=mesh,
        scratch_types=[
            pltpu.VMEM((_CHUNK,), jnp.int32),
            pltpu.VMEM((_CHUNK,), jnp.int32),
            pltpu.VMEM((_CHUNK, EMB_D), jnp.float32),
            pltpu.VMEM((_CHUNK, EMB_D), jnp.float32),
            pltpu.SemaphoreType.DMA,
            pltpu.SemaphoreType.DMA,
            pltpu.SemaphoreType.DMA,
            pltpu.SemaphoreType.DMA,
            pltpu.SemaphoreType.DMA,
        ],
    )
    def gather_kernel(idx_hbm, table_hbm, out_hbm, idx0, idx1, rows0, rows1,
                      isem0, isem1, gsem, ssem0, ssem1):
        wid = lax.axis_index("s") * _NC + lax.axis_index("c")
        base = wid * bpw
        slots = ((idx0, isem0, rows0, ssem0), (idx1, isem1, rows1, ssem1))

        def start_idx(g, slot):
            idx_v, isem, _, _ = slots[slot]
            pltpu.async_copy(idx_hbm.at[pl.ds(base + g * _CHUNK, _CHUNK)],
                             idx_v, isem)

        def step(g, slot, wait_store, prefetch):
            idx_v, isem, rows_v, ssem = slots[slot]
            off = base + g * _CHUNK
            pltpu.make_async_copy(idx_hbm.at[pl.ds(off, _CHUNK)],
                                  idx_v, isem).wait()
            if wait_store:
                pltpu.make_async_copy(rows_v, out_hbm.at[pl.ds(off, _CHUNK)],
                                      ssem).wait()
            gathers = [
                pltpu.async_copy(
                    table_hbm.at[idx_v.at[pl.ds(j * _SUB, _SUB)]],
                    rows_v.at[pl.ds(j * _SUB, _SUB)],
                    gsem,
                )
                for j in range(_CHUNK // _SUB)
            ]
            for gcp in gathers:
                gcp.wait()
            if prefetch:
                start_idx(g + 2, slot)
            pltpu.async_copy(rows_v, out_hbm.at[pl.ds(off, _CHUNK)], ssem)

        start_idx(0, 0)
        start_idx(1, 1)
        step(0, 0, wait_store=False, prefetch=True)
        step(1, 1, wait_store=False, prefetch=True)

        def loop_body(k, carry):
            g0 = 2 + 2 * k
            step(g0, 0, wait_store=True, prefetch=True)
            step(g0 + 1, 1, wait_store=True, prefetch=True)
            return carry

        lax.fori_loop(0, (n_chunks - 4) // 2, loop_body, 0)
        step(n_chunks - 2, 0, wait_store=True, prefetch=False)
        step(n_chunks - 1, 1, wait_store=True, prefetch=False)
        pltpu.make_async_copy(rows0, out_hbm.at[pl.ds(base, _CHUNK)],
                              ssem0).wait()
        pltpu.make_async_copy(rows1, out_hbm.at[pl.ds(base, _CHUNK)],
                              ssem1).wait()

    return gather_kernel


def kernel(x, atom_emb):
    b, s = x.shape
    flat_idx = x.reshape(-1).astype(jnp.int32)
    out = _make_gather(b * s)(flat_idx, atom_emb)
    return out.reshape(b, s, EMB_D)


# speedup vs baseline: 3.4578x; 1.0024x over previous
import functools

import jax
import jax.numpy as jnp
from jax import lax
from jax.experimental import pallas as pl
from jax.experimental.pallas import tpu as pltpu
from jax.experimental.pallas import tpu_sc as plsc

EMB_D = 128
_info = plsc.get_sparse_core_info()
_NC, _NS = _info.num_cores, _info.num_subcores
_NW = _NC * _NS

_CHUNK = 256
_SUB = 128


def _make_gather(n_total: int):
    assert n_total % (_NW * _CHUNK) == 0
    bpw = n_total // _NW
    n_chunks = bpw // _CHUNK
    assert n_chunks >= 4 and n_chunks % 2 == 0
    mesh = plsc.VectorSubcoreMesh(core_axis_name="c", subcore_axis_name="s")

    @functools.partial(
        pl.kernel,
        out_type=jax.ShapeDtypeStruct((n_total, EMB_D), jnp.float32),
        mesh=mesh,
        scratch_types=[
            pltpu.VMEM((_CHUNK,), jnp.int32),
            pltpu.VMEM((_CHUNK,), jnp.int32),
            pltpu.VMEM((_CHUNK, EMB_D), jnp.float32),
            pltpu.VMEM((_CHUNK, EMB_D), jnp.float32),
            pltpu.SemaphoreType.DMA,
            pltpu.SemaphoreType.DMA,
            pltpu.SemaphoreType.DMA,
            pltpu.SemaphoreType.DMA,
            pltpu.SemaphoreType.DMA,
            pltpu.SemaphoreType.DMA,
        ],
    )
    def gather_kernel(idx_hbm, table_hbm, out_hbm, idx0, idx1, rows0, rows1,
                      isem0, isem1, gsem0, gsem1, ssem0, ssem1):
        wid = lax.axis_index("s") * _NC + lax.axis_index("c")
        base = wid * bpw
        slots = ((idx0, isem0, rows0, gsem0, ssem0),
                 (idx1, isem1, rows1, gsem1, ssem1))

        def start_idx(g, slot):
            idx_v, isem = slots[slot][0], slots[slot][1]
            pltpu.async_copy(idx_hbm.at[pl.ds(base + g * _CHUNK, _CHUNK)],
                             idx_v, isem)

        def fire_gathers(g, slot, wait_store):
            idx_v, isem, rows_v, gsem, ssem = slots[slot]
            off = base + g * _CHUNK
            pltpu.make_async_copy(idx_hbm.at[pl.ds(off, _CHUNK)],
                                  idx_v, isem).wait()
            if wait_store:
                pltpu.make_async_copy(rows_v, out_hbm.at[pl.ds(off, _CHUNK)],
                                      ssem).wait()
            for j in range(_CHUNK // _SUB):
                pltpu.async_copy(
                    table_hbm.at[idx_v.at[pl.ds(j * _SUB, _SUB)]],
                    rows_v.at[pl.ds(j * _SUB, _SUB)],
                    gsem,
                )

        def retire(g, slot, prefetch):
            idx_v, isem, rows_v, gsem, ssem = slots[slot]
            off = base + g * _CHUNK
            for j in range(_CHUNK // _SUB):
                pltpu.make_async_copy(
                    table_hbm.at[idx_v.at[pl.ds(j * _SUB, _SUB)]],
                    rows_v.at[pl.ds(j * _SUB, _SUB)],
                    gsem,
                ).wait()
            if prefetch:
                start_idx(g + 2, slot)
            pltpu.async_copy(rows_v, out_hbm.at[pl.ds(off, _CHUNK)], ssem)

        start_idx(0, 0)
        start_idx(1, 1)
        fire_gathers(0, 0, wait_store=False)
        fire_gathers(1, 1, wait_store=False)
        retire(0, 0, prefetch=True)

        def pair_body(k, carry):
            g = 2 + 2 * k
            fire_gathers(g, 0, wait_store=True)
            retire(g - 1, 1, prefetch=True)
            fire_gathers(g + 1, 1, wait_store=True)
            retire(g, 0, prefetch=True)
            return carry

        lax.fori_loop(0, (n_chunks - 4) // 2, pair_body, 0)
        g = n_chunks - 2
        fire_gathers(g, 0, wait_store=True)
        retire(g - 1, 1, prefetch=True)
        fire_gathers(g + 1, 1, wait_store=True)
        retire(g, 0, prefetch=False)
        retire_last = n_chunks - 1
        idx_v, _, rows_v, gsem, ssem = slots[1]
        for j in range(_CHUNK // _SUB):
            pltpu.make_async_copy(
                table_hbm.at[idx_v.at[pl.ds(j * _SUB, _SUB)]],
                rows_v.at[pl.ds(j * _SUB, _SUB)],
                gsem,
            ).wait()
        pltpu.async_copy(
            rows_v, out_hbm.at[pl.ds(base + retire_last * _CHUNK, _CHUNK)],
            ssem)
        pltpu.make_async_copy(rows0, out_hbm.at[pl.ds(base, _CHUNK)],
                              ssem0).wait()
        pltpu.make_async_copy(rows1, out_hbm.at[pl.ds(base, _CHUNK)],
                              ssem1).wait()

    return gather_kernel


def kernel(x, atom_emb):
    b, s = x.shape
    flat_idx = x.reshape(-1).astype(jnp.int32)
    out = _make_gather(b * s)(flat_idx, atom_emb)
    return out.reshape(b, s, EMB_D)


# speedup vs baseline: 6.2813x; 1.8166x over previous
import functools

import jax
import jax.numpy as jnp
from jax import lax
from jax.experimental import pallas as pl
from jax.experimental.pallas import tpu as pltpu
from jax.experimental.pallas import tpu_sc as plsc

EMB_D = 128
_info = plsc.get_sparse_core_info()
_NC, _NS = _info.num_cores, _info.num_subcores
_NW = _NC * _NS

_R = 8
_SUB = 128


def _make_gather(n_rows: int, seq: int):
    assert n_rows % (_NW * _R) == 0
    rpw = n_rows // _NW
    n_chunks = rpw // _R
    assert n_chunks >= 4 and n_chunks % 2 == 0
    chunk_idx = _R * seq
    subs = []
    o = 0
    while o < chunk_idx:
        n = min(_SUB, chunk_idx - o)
        subs.append((o, n))
        o += n
    mesh = plsc.VectorSubcoreMesh(core_axis_name="c", subcore_axis_name="s")

    @functools.partial(
        pl.kernel,
        out_type=jax.ShapeDtypeStruct((n_rows, seq, EMB_D), jnp.float32),
        mesh=mesh,
        compiler_params=pltpu.CompilerParams(use_tc_tiling_on_sc=True),
        scratch_types=[
            pltpu.VMEM((chunk_idx,), jnp.int32),
            pltpu.VMEM((chunk_idx,), jnp.int32),
            pltpu.VMEM((chunk_idx, EMB_D), jnp.float32),
            pltpu.VMEM((chunk_idx, EMB_D), jnp.float32),
            pltpu.SemaphoreType.DMA,
            pltpu.SemaphoreType.DMA,
            pltpu.SemaphoreType.DMA,
            pltpu.SemaphoreType.DMA,
            pltpu.SemaphoreType.DMA,
            pltpu.SemaphoreType.DMA,
        ],
    )
    def gather_kernel(idx_hbm, table_hbm, out_hbm, idx0, idx1, rows0, rows1,
                      isem0, isem1, gsem0, gsem1, ssem0, ssem1):
        wid = lax.axis_index("s") * _NC + lax.axis_index("c")
        base_row = wid * rpw
        base_i = base_row * seq
        slots = ((idx0, isem0, rows0, gsem0, ssem0),
                 (idx1, isem1, rows1, gsem1, ssem1))

        def start_idx(g, slot):
            idx_v, isem = slots[slot][0], slots[slot][1]
            pltpu.async_copy(idx_hbm.at[pl.ds(base_i + g * chunk_idx,
                                              chunk_idx)], idx_v, isem)

        def drain_store(g, slot):
            _, _, rows_v, _, ssem = slots[slot]
            r0 = base_row + g * _R
            for r in range(_R):
                pltpu.make_async_copy(rows_v.at[pl.ds(r * seq, seq)],
                                      out_hbm.at[r0 + r], ssem).wait()

        def fire_gather(g, slot, wait_store):
            idx_v, isem, rows_v, gsem, ssem = slots[slot]
            pltpu.make_async_copy(
                idx_hbm.at[pl.ds(base_i + g * chunk_idx, chunk_idx)],
                idx_v, isem).wait()
            if wait_store:
                drain_store(g, slot)
            for (o, n) in subs:
                pltpu.async_copy(table_hbm.at[idx_v.at[pl.ds(o, n)]],
                                 rows_v.at[pl.ds(o, n)], gsem)

        def retire(g, slot, prefetch):
            idx_v, isem, rows_v, gsem, ssem = slots[slot]
            r0 = base_row + g * _R
            for (o, n) in subs:
                pltpu.make_async_copy(table_hbm.at[idx_v.at[pl.ds(o, n)]],
                                      rows_v.at[pl.ds(o, n)], gsem).wait()
            if prefetch:
                start_idx(g + 2, slot)
            for r in range(_R):
                pltpu.async_copy(rows_v.at[pl.ds(r * seq, seq)],
                                 out_hbm.at[r0 + r], ssem)

        start_idx(0, 0)
        start_idx(1, 1)
        fire_gather(0, 0, wait_store=False)
        fire_gather(1, 1, wait_store=False)
        retire(0, 0, prefetch=True)

        def pair_body(k, carry):
            g = 2 + 2 * k
            fire_gather(g, 0, wait_store=True)
            retire(g - 1, 1, prefetch=True)
            fire_gather(g + 1, 1, wait_store=True)
            retire(g, 0, prefetch=True)
            return carry

        lax.fori_loop(0, (n_chunks - 4) // 2, pair_body, 0)
        g = n_chunks - 2
        fire_gather(g, 0, wait_store=True)
        retire(g - 1, 1, prefetch=True)
        fire_gather(g + 1, 1, wait_store=True)
        retire(g, 0, prefetch=False)
        retire(n_chunks - 1, 1, prefetch=False)
        drain_store(n_chunks - 2, 0)
        drain_store(n_chunks - 1, 1)

    return gather_kernel


def kernel(x, atom_emb):
    b, s = x.shape
    flat_idx = x.reshape(-1).astype(jnp.int32)
    return _make_gather(b, s)(flat_idx, atom_emb)


# speedup vs baseline: 6.2848x; 1.0006x over previous
import functools

import jax
import jax.numpy as jnp
from jax import lax
from jax.experimental import pallas as pl
from jax.experimental.pallas import tpu as pltpu
from jax.experimental.pallas import tpu_sc as plsc

EMB_D = 128
_info = plsc.get_sparse_core_info()
_NC, _NS = _info.num_cores, _info.num_subcores
_NW = _NC * _NS

_R = 8
_SUB = 128


def _make_gather(n_rows: int, seq: int):
    assert n_rows % (_NW * _R) == 0
    rpw = n_rows // _NW
    n_chunks = rpw // _R
    assert n_chunks >= 4 and n_chunks % 2 == 0
    chunk_idx = _R * seq
    subs = []
    o = 0
    while o < chunk_idx:
        n = min(_SUB, chunk_idx - o)
        subs.append((o, n))
        o += n
    mesh = plsc.VectorSubcoreMesh(core_axis_name="c", subcore_axis_name="s")

    @functools.partial(
        pl.kernel,
        out_type=jax.ShapeDtypeStruct((n_rows, seq, EMB_D), jnp.float32),
        mesh=mesh,
        compiler_params=pltpu.CompilerParams(use_tc_tiling_on_sc=True,
                                             needs_layout_passes=True),
        scratch_types=[
            pltpu.VMEM((chunk_idx,), jnp.int32),
            pltpu.VMEM((chunk_idx,), jnp.int32),
            pltpu.VMEM((chunk_idx, EMB_D), jnp.float32),
            pltpu.VMEM((chunk_idx, EMB_D), jnp.float32),
            pltpu.SemaphoreType.DMA,
            pltpu.SemaphoreType.DMA,
            pltpu.SemaphoreType.DMA,
            pltpu.SemaphoreType.DMA,
            pltpu.SemaphoreType.DMA,
            pltpu.SemaphoreType.DMA,
        ],
    )
    def gather_kernel(idx_hbm, table_hbm, out_hbm, idx0, idx1, rows0, rows1,
                      isem0, isem1, gsem0, gsem1, ssem0, ssem1):
        wid = lax.axis_index("s") * _NC + lax.axis_index("c")
        base_row = wid * rpw
        base_i = base_row * seq
        slots = ((idx0, isem0, rows0, gsem0, ssem0),
                 (idx1, isem1, rows1, gsem1, ssem1))

        def start_idx(g, slot):
            idx_v, isem = slots[slot][0], slots[slot][1]
            pltpu.async_copy(idx_hbm.at[pl.ds(base_i + g * chunk_idx,
                                              chunk_idx)], idx_v, isem)

        def drain_store(g, slot):
            _, _, rows_v, _, ssem = slots[slot]
            r0 = base_row + g * _R
            for r in range(_R):
                pltpu.make_async_copy(rows_v.at[pl.ds(r * seq, seq)],
                                      out_hbm.at[r0 + r], ssem).wait()

        def fire_gather(g, slot, wait_store):
            idx_v, isem, rows_v, gsem, ssem = slots[slot]
            pltpu.make_async_copy(
                idx_hbm.at[pl.ds(base_i + g * chunk_idx, chunk_idx)],
                idx_v, isem).wait()
            if wait_store:
                drain_store(g, slot)
            for (o, n) in subs:
                pltpu.async_copy(table_hbm.at[idx_v.at[pl.ds(o, n)]],
                                 rows_v.at[pl.ds(o, n)], gsem)

        def retire(g, slot, prefetch):
            idx_v, isem, rows_v, gsem, ssem = slots[slot]
            r0 = base_row + g * _R
            for (o, n) in subs:
                pltpu.make_async_copy(table_hbm.at[idx_v.at[pl.ds(o, n)]],
                                      rows_v.at[pl.ds(o, n)], gsem).wait()
            if prefetch:
                start_idx(g + 2, slot)
            for r in range(_R):
                pltpu.async_copy(rows_v.at[pl.ds(r * seq, seq)],
                                 out_hbm.at[r0 + r], ssem)

        start_idx(0, 0)
        start_idx(1, 1)
        fire_gather(0, 0, wait_store=False)
        fire_gather(1, 1, wait_store=False)
        retire(0, 0, prefetch=True)

        def pair_body(k, carry):
            g = 2 + 2 * k
            fire_gather(g, 0, wait_store=True)
            retire(g - 1, 1, prefetch=True)
            fire_gather(g + 1, 1, wait_store=True)
            retire(g, 0, prefetch=True)
            return carry

        lax.fori_loop(0, (n_chunks - 4) // 2, pair_body, 0)
        g = n_chunks - 2
        fire_gather(g, 0, wait_store=True)
        retire(g - 1, 1, prefetch=True)
        fire_gather(g + 1, 1, wait_store=True)
        retire(g, 0, prefetch=False)
        retire(n_chunks - 1, 1, prefetch=False)
        drain_store(n_chunks - 2, 0)
        drain_store(n_chunks - 1, 1)

    return gather_kernel


def kernel(x, atom_emb):
    b, s = x.shape
    flat_idx = x.reshape(-1).astype(jnp.int32)
    return _make_gather(b, s)(flat_idx, atom_emb)


# speedup vs baseline: 11.9318x; 1.8985x over previous
import functools

import jax
import jax.numpy as jnp
from jax import lax
from jax.experimental import pallas as pl
from jax.experimental.pallas import tpu as pltpu
from jax.experimental.pallas import tpu_sc as plsc

EMB_D = 128
_info = plsc.get_sparse_core_info()
_NC, _NS = _info.num_cores, _info.num_subcores
_NW = _NC * _NS

_CHUNK = 400
_SUB = 128


def _make_gather(n_total: int):
    assert n_total % (_NW * _CHUNK) == 0
    bpw = n_total // _NW
    n_chunks = bpw // _CHUNK
    assert n_chunks >= 4 and n_chunks % 2 == 0
    subs = []
    o = 0
    while o < _CHUNK:
        n = min(_SUB, _CHUNK - o)
        subs.append((o, n))
        o += n
    mesh = plsc.VectorSubcoreMesh(core_axis_name="c", subcore_axis_name="s")

    @functools.partial(
        pl.kernel,
        out_type=jax.ShapeDtypeStruct((n_total, EMB_D), jnp.float32),
        mesh=mesh,
        compiler_params=pltpu.CompilerParams(use_tc_tiling_on_sc=True),
        scratch_types=[
            pltpu.VMEM((_CHUNK,), jnp.int32),
            pltpu.VMEM((_CHUNK,), jnp.int32),
            pltpu.VMEM((_CHUNK, EMB_D), jnp.float32),
            pltpu.VMEM((_CHUNK, EMB_D), jnp.float32),
            pltpu.SemaphoreType.DMA,
            pltpu.SemaphoreType.DMA,
            pltpu.SemaphoreType.DMA,
            pltpu.SemaphoreType.DMA,
            pltpu.SemaphoreType.DMA,
            pltpu.SemaphoreType.DMA,
        ],
    )
    def gather_kernel(idx_hbm, table_hbm, out_hbm, idx0, idx1, rows0, rows1,
                      isem0, isem1, gsem0, gsem1, ssem0, ssem1):
        wid = lax.axis_index("s") * _NC + lax.axis_index("c")
        base = wid * bpw
        slots = ((idx0, isem0, rows0, gsem0, ssem0),
                 (idx1, isem1, rows1, gsem1, ssem1))

        def start_idx(g, slot):
            idx_v, isem = slots[slot][0], slots[slot][1]
            pltpu.async_copy(idx_hbm.at[pl.ds(base + g * _CHUNK, _CHUNK)],
                             idx_v, isem)

        def fire_gather(g, slot, wait_store):
            idx_v, isem, rows_v, gsem, ssem = slots[slot]
            off = base + g * _CHUNK
            pltpu.make_async_copy(idx_hbm.at[pl.ds(off, _CHUNK)],
                                  idx_v, isem).wait()
            if wait_store:
                pltpu.make_async_copy(rows_v, out_hbm.at[pl.ds(off, _CHUNK)],
                                      ssem).wait()
            for (o, n) in subs:
                pltpu.async_copy(table_hbm.at[idx_v.at[pl.ds(o, n)]],
                                 rows_v.at[pl.ds(o, n)], gsem)

        def retire(g, slot, prefetch):
            idx_v, isem, rows_v, gsem, ssem = slots[slot]
            off = base + g * _CHUNK
            for (o, n) in subs:
                pltpu.make_async_copy(table_hbm.at[idx_v.at[pl.ds(o, n)]],
                                      rows_v.at[pl.ds(o, n)], gsem).wait()
            if prefetch:
                start_idx(g + 2, slot)
            pltpu.async_copy(rows_v, out_hbm.at[pl.ds(off, _CHUNK)], ssem)

        start_idx(0, 0)
        start_idx(1, 1)
        fire_gather(0, 0, wait_store=False)
        fire_gather(1, 1, wait_store=False)
        retire(0, 0, prefetch=True)

        def pair_body(k, carry):
            g = 2 + 2 * k
            fire_gather(g, 0, wait_store=True)
            retire(g - 1, 1, prefetch=True)
            fire_gather(g + 1, 1, wait_store=True)
            retire(g, 0, prefetch=True)
            return carry

        lax.fori_loop(0, (n_chunks - 4) // 2, pair_body, 0)
        g = n_chunks - 2
        fire_gather(g, 0, wait_store=True)
        retire(g - 1, 1, prefetch=True)
        fire_gather(g + 1, 1, wait_store=True)
        retire(g, 0, prefetch=False)
        retire(n_chunks - 1, 1, prefetch=False)
        pltpu.make_async_copy(rows0, out_hbm.at[pl.ds(base, _CHUNK)],
                              ssem0).wait()
        pltpu.make_async_copy(rows1, out_hbm.at[pl.ds(base, _CHUNK)],
                              ssem1).wait()

    return gather_kernel


def kernel(x, atom_emb):
    b, s = x.shape
    perm_idx = x.T.reshape(-1).astype(jnp.int32)
    out2d = _make_gather(b * s)(perm_idx, atom_emb)
    return out2d.reshape(s, b, EMB_D).transpose(1, 0, 2)


# speedup vs baseline: 11.9455x; 1.0011x over previous
import functools

import jax
import jax.numpy as jnp
from jax import lax
from jax.experimental import pallas as pl
from jax.experimental.pallas import tpu as pltpu
from jax.experimental.pallas import tpu_sc as plsc

EMB_D = 128
_info = plsc.get_sparse_core_info()
_NC, _NS = _info.num_cores, _info.num_subcores
_NW = _NC * _NS

_CHUNK = 400
_SUB = 400


def _make_gather(n_total: int):
    assert n_total % (_NW * _CHUNK) == 0
    bpw = n_total // _NW
    n_chunks = bpw // _CHUNK
    assert n_chunks >= 4 and n_chunks % 2 == 0
    subs = []
    o = 0
    while o < _CHUNK:
        n = min(_SUB, _CHUNK - o)
        subs.append((o, n))
        o += n
    mesh = plsc.VectorSubcoreMesh(core_axis_name="c", subcore_axis_name="s")

    @functools.partial(
        pl.kernel,
        out_type=jax.ShapeDtypeStruct((n_total, EMB_D), jnp.float32),
        mesh=mesh,
        compiler_params=pltpu.CompilerParams(use_tc_tiling_on_sc=True),
        scratch_types=[
            pltpu.VMEM((_CHUNK,), jnp.int32),
            pltpu.VMEM((_CHUNK,), jnp.int32),
            pltpu.VMEM((_CHUNK, EMB_D), jnp.float32),
            pltpu.VMEM((_CHUNK, EMB_D), jnp.float32),
            pltpu.SemaphoreType.DMA,
            pltpu.SemaphoreType.DMA,
            pltpu.SemaphoreType.DMA,
            pltpu.SemaphoreType.DMA,
            pltpu.SemaphoreType.DMA,
            pltpu.SemaphoreType.DMA,
        ],
    )
    def gather_kernel(idx_hbm, table_hbm, out_hbm, idx0, idx1, rows0, rows1,
                      isem0, isem1, gsem0, gsem1, ssem0, ssem1):
        wid = lax.axis_index("s") * _NC + lax.axis_index("c")
        base = wid * bpw
        slots = ((idx0, isem0, rows0, gsem0, ssem0),
                 (idx1, isem1, rows1, gsem1, ssem1))

        def start_idx(g, slot):
            idx_v, isem = slots[slot][0], slots[slot][1]
            pltpu.async_copy(idx_hbm.at[pl.ds(base + g * _CHUNK, _CHUNK)],
                             idx_v, isem)

        def fire_gather(g, slot, wait_store):
            idx_v, isem, rows_v, gsem, ssem = slots[slot]
            off = base + g * _CHUNK
            pltpu.make_async_copy(idx_hbm.at[pl.ds(off, _CHUNK)],
                                  idx_v, isem).wait()
            if wait_store:
                pltpu.make_async_copy(rows_v, out_hbm.at[pl.ds(off, _CHUNK)],
                                      ssem).wait()
            for (o, n) in subs:
                pltpu.async_copy(table_hbm.at[idx_v.at[pl.ds(o, n)]],
                                 rows_v.at[pl.ds(o, n)], gsem)

        def retire(g, slot, prefetch):
            idx_v, isem, rows_v, gsem, ssem = slots[slot]
            off = base + g * _CHUNK
            for (o, n) in subs:
                pltpu.make_async_copy(table_hbm.at[idx_v.at[pl.ds(o, n)]],
                                      rows_v.at[pl.ds(o, n)], gsem).wait()
            if prefetch:
                start_idx(g + 2, slot)
            pltpu.async_copy(rows_v, out_hbm.at[pl.ds(off, _CHUNK)], ssem)

        start_idx(0, 0)
        start_idx(1, 1)
        fire_gather(0, 0, wait_store=False)
        fire_gather(1, 1, wait_store=False)
        retire(0, 0, prefetch=True)

        def pair_body(k, carry):
            g = 2 + 2 * k
            fire_gather(g, 0, wait_store=True)
            retire(g - 1, 1, prefetch=True)
            fire_gather(g + 1, 1, wait_store=True)
            retire(g, 0, prefetch=True)
            return carry

        lax.fori_loop(0, (n_chunks - 4) // 2, pair_body, 0)
        g = n_chunks - 2
        fire_gather(g, 0, wait_store=True)
        retire(g - 1, 1, prefetch=True)
        fire_gather(g + 1, 1, wait_store=True)
        retire(g, 0, prefetch=False)
        retire(n_chunks - 1, 1, prefetch=False)
        pltpu.make_async_copy(rows0, out_hbm.at[pl.ds(base, _CHUNK)],
                              ssem0).wait()
        pltpu.make_async_copy(rows1, out_hbm.at[pl.ds(base, _CHUNK)],
                              ssem1).wait()

    return gather_kernel


def kernel(x, atom_emb):
    b, s = x.shape
    perm_idx = x.T.reshape(-1).astype(jnp.int32)
    out2d = _make_gather(b * s)(perm_idx, atom_emb)
    return out2d.reshape(s, b, EMB_D).transpose(1, 0, 2)
